# baseline (speedup 1.0000x reference)
import jax
import jax.numpy as jnp
from jax.experimental import pallas as pl
from jax.experimental.pallas import tpu as pltpu

P = 64
N = 1024
HD = 8
OC = 64
D = HD * OC
NH = N // 2

_F32 = jnp.float32
_BF16 = jnp.bfloat16


def _iota(shape, dim):
    return jax.lax.broadcasted_iota(jnp.int32, shape, dim)


def _expand_k():
    return jnp.where(_iota((HD, D), 0) == _iota((HD, D), 1) // OC,
                     1.0, 0.0).astype(_BF16)


def _blocksum_m():
    return jnp.where(_iota((D, HD), 0) // OC == _iota((D, HD), 1),
                     1.0, 0.0).astype(_BF16)


def _sel_p():
    return jnp.where(_iota((P, D), 0) == _iota((P, D), 1) % OC,
                     1.0, 0.0).astype(_BF16)


def _sel_p_t():
    return jnp.where(_iota((D, P), 0) % OC == _iota((D, P), 1),
                     1.0, 0.0).astype(_BF16)


def _blockmask(dtype):
    return jnp.where(_iota((D, D), 0) // OC == _iota((D, D), 1) // OC,
                     1.0, 0.0).astype(dtype)


def _mm(a, b):
    return jax.lax.dot_general(a.astype(_BF16), b.astype(_BF16),
                               (((1,), (0,)), ((), ())),
                               preferred_element_type=_F32)


def _mm_t(a, b):
    return jax.lax.dot_general(a.astype(_BF16), b.astype(_BF16),
                               (((1,), (1,)), ((), ())),
                               preferred_element_type=_F32)


def _rep64(a):
    return _mm(a, _expand_k())


def _leaky(v):
    return jnp.where(v >= 0, v, 0.2 * v)


def _node_att_prep(h_prox, asd):
    sc_prox = _mm(h_prox, asd)
    as_prox = sc_prox[:, :HD]
    r1 = _rep64(as_prox)
    v = jnp.sum(r1 * _sel_p().astype(_F32), axis=0, keepdims=True)
    h_bd = _mm(_sel_p_t(), h_prox) * _blockmask(_F32)
    return sc_prox, v, h_bd


def _node_attention(v, h_bd, h_n, sc_n):
    as_n, ad_n = sc_n[:, :HD], sc_n[:, HD:]
    e_node = _leaky(v + _rep64(ad_n))
    ex_node = jnp.exp(e_node)
    ex_self = jnp.exp(_leaky(as_n + ad_n))
    denom = _mm(ex_node, _blocksum_m()) + ex_self + 1e-16
    alpha = ex_node / _rep64(denom)
    return _mm(alpha, h_bd) + _rep64(ex_self / denom) * h_n


def _gat_layer(h_prox, h_nodes, asd, need_prox):
    sc_prox, v, h_bd = _node_att_prep(h_prox, asd)
    sc_nodes = _mm(h_nodes, asd)
    as_prox, ad_prox = sc_prox[:, :HD], sc_prox[:, HD:]
    as_nodes = sc_nodes[:, :HD]

    out_nodes = _node_attention(v, h_bd, h_nodes, sc_nodes)

    if not need_prox:
        return None, out_nodes

    as_stack = _mm_t(_blocksum_m(), as_nodes)
    rowsel = jnp.where(_iota((D, P), 0) % OC == _iota((D, P), 1),
                       1.0, 0.0).astype(_F32)
    y_d = _mm_t(_blocksum_m(), ad_prox)
    ad_prox_col = jnp.sum(y_d * rowsel, axis=1, keepdims=True)
    y_s = _mm_t(_blocksum_m(), as_prox)
    as_prox_col = jnp.sum(y_s * rowsel, axis=1, keepdims=True)

    ex_prox = jnp.exp(_leaky(as_stack + ad_prox_col))
    ex_self_p = jnp.exp(_leaky(as_prox_col + ad_prox_col))
    denom_p = (jnp.sum(ex_prox, axis=1, keepdims=True)
               + ex_self_p + 1e-16)
    alpha_p = ex_prox / denom_p
    r_full = _mm(alpha_p, h_nodes)
    out_prox = _mm(_sel_p(), r_full * _blockmask(_F32))
    alpha_self_col = ex_self_p / denom_p
    rep_self = _mm(_sel_p(), alpha_self_col * _blockmask(_F32))
    out_prox = out_prox + rep_self * h_prox
    return out_prox, out_nodes


def _att_blockdiag_in(att_s, att_d):
    sel_k = _blocksum_m()
    rowsel = jnp.where(_iota((D, OC), 0) % OC == _iota((D, OC), 1),
                       1.0, 0.0).astype(_F32)
    s_col = jnp.sum(_mm(sel_k, att_s) * rowsel, axis=1, keepdims=True)
    d_col = jnp.sum(_mm(sel_k, att_d) * rowsel, axis=1, keepdims=True)
    sel_f = sel_k.astype(_F32)
    return jnp.concatenate([s_col * sel_f, d_col * sel_f], axis=1)


def _model_body(x_ref, prox_ref, W1_ref,
                as1_ref, ad1_ref, as2_ref, ad2_ref,
                W2_ref, Wm1_ref, Wm2_ref, Wf_ref,
                b1_ref, b2_ref, bm1_ref, bm2_ref, bf_ref,
                preds_ref, feats_ref,
                x_s, w2_s, wm1_s, wm2_s, wf_s, f3_s, preds_s,
                sem_x, sem_w2, sem_wm1, sem_wm2, sem_wf,
                sem_f3, sem_p1, sem_p2):
    cp_x = pltpu.make_async_copy(x_ref, x_s, sem_x)
    cp_w2 = pltpu.make_async_copy(W2_ref, w2_s, sem_w2)
    cp_wm1 = pltpu.make_async_copy(Wm1_ref, wm1_s, sem_wm1)
    cp_wm2 = pltpu.make_async_copy(Wm2_ref, wm2_s, sem_wm2)
    cp_wf = pltpu.make_async_copy(Wf_ref, wf_s, sem_wf)
    cp_x.start()
    cp_w2.start()
    cp_wm1.start()
    cp_wm2.start()
    cp_wf.start()

    asd1 = _att_blockdiag_in(as1_ref[...], ad1_ref[...])
    asd2 = _att_blockdiag_in(as2_ref[...], ad2_ref[...])
    h1_prox = _mm(prox_ref[...], W1_ref[...])
    cp_x.wait()
    h1_nodes = _mm(x_s[...], W1_ref[...])
    p1, n1 = _gat_layer(h1_prox, h1_nodes, asd1, need_prox=True)
    f1_prox = jax.nn.relu(p1 + b1_ref[...])
    f1_nodes = jax.nn.relu(n1 + b1_ref[...])

    cp_w2.wait()
    h2_prox = _mm(f1_prox, w2_s[...])
    sc2_prox, v2, h2_bd = _node_att_prep(h2_prox, asd2)

    out_copies = []
    for g in range(2):
        rows = slice(g * NH, (g + 1) * NH)
        h2n = _mm(f1_nodes[rows], w2_s[...])
        sc2n = _mm(h2n, asd2)
        n2 = _node_attention(v2, h2_bd, h2n, sc2n)
        f2h = jax.nn.relu(n2 + b2_ref[...])
        if g == 0:
            cp_wm1.wait()
        hmid = jax.nn.relu(_mm(f2h, wm1_s[...]) + bm1_ref[...])
        if g == 0:
            cp_wm2.wait()
        f3h = jax.nn.relu(_mm(hmid, wm2_s[...]) + bm2_ref[...])
        f3_s[rows, :] = f3h
        cp_f = pltpu.make_async_copy(f3_s.at[rows, :],
                                     feats_ref.at[rows, :],
                                     sem_f3 if g == 0 else sem_p1)
        cp_f.start()
        if g == 0:
            cp_wf.wait()
        predsh = _mm(f3h, wf_s[...]) + bf_ref[...]
        preds_s[rows, :] = predsh
        cp_p = pltpu.make_async_copy(preds_s.at[rows, :],
                                     preds_ref.at[rows, :],
                                     sem_wf if g == 0 else sem_p2)
        cp_p.start()
        out_copies += [cp_f, cp_p]

    for cp in out_copies:
        cp.wait()


def kernel(x, proxies, W1, as1, ad1, b1, W2, as2, ad2, b2,
           Wm1, bm1, Wm2, bm2, Wf, bf):
    out_shape = (jax.ShapeDtypeStruct((N, D), _F32),
                 jax.ShapeDtypeStruct((N, D), _F32))
    vm = pl.BlockSpec(memory_space=pltpu.VMEM)
    hbm = pl.BlockSpec(memory_space=pltpu.HBM)
    preds, feats = pl.pallas_call(
        _model_body,
        out_shape=out_shape,
        in_specs=[hbm, vm, vm,
                  vm, vm, vm, vm,
                  hbm, hbm, hbm, hbm,
                  vm, vm, vm, vm, vm],
        out_specs=(hbm, hbm),
        scratch_shapes=[
            pltpu.VMEM((N, D), _F32),
            pltpu.VMEM((D, D), _F32),
            pltpu.VMEM((D, 4 * D), _F32),
            pltpu.VMEM((4 * D, D), _F32),
            pltpu.VMEM((D, D), _F32),
            pltpu.VMEM((N, D), _F32),
            pltpu.VMEM((N, D), _F32),
            pltpu.SemaphoreType.DMA,
            pltpu.SemaphoreType.DMA,
            pltpu.SemaphoreType.DMA,
            pltpu.SemaphoreType.DMA,
            pltpu.SemaphoreType.DMA,
            pltpu.SemaphoreType.DMA,
            pltpu.SemaphoreType.DMA,
            pltpu.SemaphoreType.DMA,
        ],
    )(x, proxies, W1,
      as1, ad1, as2, ad2,
      W2, Wm1, Wm2, Wf,
      b1.reshape(1, D), b2.reshape(1, D),
      bm1.reshape(1, 4 * D), bm2.reshape(1, D), bf.reshape(1, D))
    return preds, feats


# speedup vs baseline: 1.0826x; 1.0826x over previous
import jax
import jax.numpy as jnp
from jax.experimental import pallas as pl
from jax.experimental.pallas import tpu as pltpu

P = 64
N = 1024
HD = 8
OC = 64
D = HD * OC
NH = N // 2

_F32 = jnp.float32
_BF16 = jnp.bfloat16


def _iota(shape, dim):
    return jax.lax.broadcasted_iota(jnp.int32, shape, dim)


def _expand_k():
    return jnp.where(_iota((HD, D), 0) == _iota((HD, D), 1) // OC,
                     1.0, 0.0).astype(_BF16)


def _blocksum_m():
    return jnp.where(_iota((D, HD), 0) // OC == _iota((D, HD), 1),
                     1.0, 0.0).astype(_BF16)


def _sel_p():
    return jnp.where(_iota((P, D), 0) == _iota((P, D), 1) % OC,
                     1.0, 0.0).astype(_BF16)


def _sel_p_t():
    return jnp.where(_iota((D, P), 0) % OC == _iota((D, P), 1),
                     1.0, 0.0).astype(_BF16)


def _blockmask(dtype):
    return jnp.where(_iota((D, D), 0) // OC == _iota((D, D), 1) // OC,
                     1.0, 0.0).astype(dtype)


def _mm(a, b):
    return jax.lax.dot_general(a.astype(_BF16), b.astype(_BF16),
                               (((1,), (0,)), ((), ())),
                               preferred_element_type=_F32)


def _mm_t(a, b):
    return jax.lax.dot_general(a.astype(_BF16), b.astype(_BF16),
                               (((1,), (1,)), ((), ())),
                               preferred_element_type=_F32)


def _rep64(a):
    return _mm(a, _expand_k())


def _leaky(v):
    return jnp.where(v >= 0, v, 0.2 * v)


def _gat_layer(h_prox, h_nodes, asd, need_prox):
    sc_prox = _mm(h_prox, asd)
    sc_nodes = _mm(h_nodes, asd)
    as_prox, ad_prox = sc_prox[:, :HD], sc_prox[:, HD:]
    as_nodes, ad_nodes = sc_nodes[:, :HD], sc_nodes[:, HD:]

    r1 = _rep64(as_prox)
    v = jnp.sum(r1 * _sel_p().astype(_F32), axis=0, keepdims=True)
    e_node = _leaky(v + _rep64(ad_nodes))
    ex_node = jnp.exp(e_node)
    ex_self_n = jnp.exp(_leaky(as_nodes + ad_nodes))
    denom_n = _mm(ex_node, _blocksum_m()) + ex_self_n + 1e-16
    alpha_n = ex_node / _rep64(denom_n)
    alpha_self_n = ex_self_n / denom_n
    h_bd = (_mm(_sel_p_t(), h_prox) * _blockmask(_F32))
    out_nodes = (_mm(alpha_n, h_bd)
                 + _rep64(alpha_self_n) * h_nodes)

    if not need_prox:
        return None, out_nodes

    as_stack = _mm_t(_blocksum_m(), as_nodes)
    rowsel = jnp.where(_iota((D, P), 0) % OC == _iota((D, P), 1),
                       1.0, 0.0).astype(_F32)
    y_d = _mm_t(_blocksum_m(), ad_prox)
    ad_prox_col = jnp.sum(y_d * rowsel, axis=1, keepdims=True)
    y_s = _mm_t(_blocksum_m(), as_prox)
    as_prox_col = jnp.sum(y_s * rowsel, axis=1, keepdims=True)

    ex_prox = jnp.exp(_leaky(as_stack + ad_prox_col))
    ex_self_p = jnp.exp(_leaky(as_prox_col + ad_prox_col))
    denom_p = (jnp.sum(ex_prox, axis=1, keepdims=True)
               + ex_self_p + 1e-16)
    alpha_p = ex_prox / denom_p
    r_full = _mm(alpha_p, h_nodes)
    out_prox = _mm(_sel_p(), r_full * _blockmask(_F32))
    alpha_self_col = ex_self_p / denom_p
    rep_self = _mm(_sel_p(), alpha_self_col * _blockmask(_F32))
    out_prox = out_prox + rep_self * h_prox
    return out_prox, out_nodes


def _att_blockdiag_in(att_s, att_d):
    sel_k = _blocksum_m()
    rowsel = jnp.where(_iota((D, OC), 0) % OC == _iota((D, OC), 1),
                       1.0, 0.0).astype(_F32)
    s_col = jnp.sum(_mm(sel_k, att_s) * rowsel, axis=1, keepdims=True)
    d_col = jnp.sum(_mm(sel_k, att_d) * rowsel, axis=1, keepdims=True)
    sel_f = sel_k.astype(_F32)
    return jnp.concatenate([s_col * sel_f, d_col * sel_f], axis=1)


def _model_body(x_ref, prox_ref, W1_ref,
                as1_ref, ad1_ref, as2_ref, ad2_ref,
                W2_ref, Wm1_ref, Wm2_ref, Wf_ref,
                b1_ref, b2_ref, bm1_ref, bm2_ref, bf_ref,
                preds_ref, feats_ref,
                x_s, w2_s, wm1_s, wm2_s, wf_s, f3_s, preds_s,
                sem_x, sem_w2, sem_wm1, sem_wm2, sem_wf,
                sem_f3, sem_p1, sem_p2):
    cp_x = pltpu.make_async_copy(x_ref, x_s, sem_x)
    cp_w2 = pltpu.make_async_copy(W2_ref, w2_s, sem_w2)
    cp_wm1 = pltpu.make_async_copy(Wm1_ref, wm1_s, sem_wm1)
    cp_wm2 = pltpu.make_async_copy(Wm2_ref, wm2_s, sem_wm2)
    cp_wf = pltpu.make_async_copy(Wf_ref, wf_s, sem_wf)
    cp_x.start()
    cp_w2.start()
    cp_wm1.start()
    cp_wm2.start()
    cp_wf.start()

    asd1 = _att_blockdiag_in(as1_ref[...], ad1_ref[...])
    asd2 = _att_blockdiag_in(as2_ref[...], ad2_ref[...])
    h1_prox = _mm(prox_ref[...], W1_ref[...])
    cp_x.wait()
    h1_nodes = _mm(x_s[...], W1_ref[...])
    p1, n1 = _gat_layer(h1_prox, h1_nodes, asd1, need_prox=True)
    f1_prox = jax.nn.relu(p1 + b1_ref[...])
    f1_nodes = jax.nn.relu(n1 + b1_ref[...])

    cp_w2.wait()
    h2_prox = _mm(f1_prox, w2_s[...])
    h2_nodes = _mm(f1_nodes, w2_s[...])
    _, n2 = _gat_layer(h2_prox, h2_nodes, asd2, need_prox=False)
    f2 = jax.nn.relu(n2 + b2_ref[...])

    cp_wm1.wait()
    hmid = jax.nn.relu(_mm(f2, wm1_s[...]) + bm1_ref[...])
    cp_wm2.wait()
    f3 = jax.nn.relu(_mm(hmid, wm2_s[...]) + bm2_ref[...])
    f3_s[...] = f3
    cp_f3 = pltpu.make_async_copy(f3_s, feats_ref, sem_f3)
    cp_f3.start()

    cp_wf.wait()
    fh = f3.astype(_BF16)
    wfb = wf_s[...].astype(_BF16)
    preds_s[0:NH, :] = jax.lax.dot_general(
        fh[0:NH, :], wfb, (((1,), (0,)), ((), ())),
        preferred_element_type=_F32) + bf_ref[...]
    cp_p1 = pltpu.make_async_copy(preds_s.at[0:NH, :],
                                  preds_ref.at[0:NH, :], sem_p1)
    cp_p1.start()
    preds_s[NH:N, :] = jax.lax.dot_general(
        fh[NH:N, :], wfb, (((1,), (0,)), ((), ())),
        preferred_element_type=_F32) + bf_ref[...]
    cp_p2 = pltpu.make_async_copy(preds_s.at[NH:N, :],
                                  preds_ref.at[NH:N, :], sem_p2)
    cp_p2.start()
    cp_f3.wait()
    cp_p1.wait()
    cp_p2.wait()


def kernel(x, proxies, W1, as1, ad1, b1, W2, as2, ad2, b2,
           Wm1, bm1, Wm2, bm2, Wf, bf):
    out_shape = (jax.ShapeDtypeStruct((N, D), _F32),
                 jax.ShapeDtypeStruct((N, D), _F32))
    vm = pl.BlockSpec(memory_space=pltpu.VMEM)
    hbm = pl.BlockSpec(memory_space=pltpu.HBM)
    preds, feats = pl.pallas_call(
        _model_body,
        out_shape=out_shape,
        in_specs=[hbm, vm, vm,
                  vm, vm, vm, vm,
                  hbm, hbm, hbm, hbm,
                  vm, vm, vm, vm, vm],
        out_specs=(hbm, hbm),
        scratch_shapes=[
            pltpu.VMEM((N, D), _F32),
            pltpu.VMEM((D, D), _F32),
            pltpu.VMEM((D, 4 * D), _F32),
            pltpu.VMEM((4 * D, D), _F32),
            pltpu.VMEM((D, D), _F32),
            pltpu.VMEM((N, D), _F32),
            pltpu.VMEM((N, D), _F32),
            pltpu.SemaphoreType.DMA,
            pltpu.SemaphoreType.DMA,
            pltpu.SemaphoreType.DMA,
            pltpu.SemaphoreType.DMA,
            pltpu.SemaphoreType.DMA,
            pltpu.SemaphoreType.DMA,
            pltpu.SemaphoreType.DMA,
            pltpu.SemaphoreType.DMA,
        ],
    )(x, proxies, W1,
      as1, ad1, as2, ad2,
      W2, Wm1, Wm2, Wf,
      b1.reshape(1, D), b2.reshape(1, D),
      bm1.reshape(1, 4 * D), bm2.reshape(1, D), bf.reshape(1, D))
    return preds, feats


# speedup vs baseline: 1.1213x; 1.0357x over previous
import jax
import jax.numpy as jnp
from jax.experimental import pallas as pl
from jax.experimental.pallas import tpu as pltpu

P = 64
N = 1024
HD = 8
OC = 64
D = HD * OC
NH = N // 2

_F32 = jnp.float32
_BF16 = jnp.bfloat16


def _iota(shape, dim):
    return jax.lax.broadcasted_iota(jnp.int32, shape, dim)


def _expand_k():
    return jnp.where(_iota((HD, D), 0) == _iota((HD, D), 1) // OC,
                     1.0, 0.0).astype(_BF16)


def _blocksum_m():
    return jnp.where(_iota((D, HD), 0) // OC == _iota((D, HD), 1),
                     1.0, 0.0).astype(_BF16)


def _sel_p():
    return jnp.where(_iota((P, D), 0) == _iota((P, D), 1) % OC,
                     1.0, 0.0).astype(_BF16)


def _sel_p_t():
    return jnp.where(_iota((D, P), 0) % OC == _iota((D, P), 1),
                     1.0, 0.0).astype(_BF16)


def _blockmask(dtype):
    return jnp.where(_iota((D, D), 0) // OC == _iota((D, D), 1) // OC,
                     1.0, 0.0).astype(dtype)


def _mm(a, b):
    return jax.lax.dot_general(a.astype(_BF16), b.astype(_BF16),
                               (((1,), (0,)), ((), ())),
                               preferred_element_type=_F32)


def _mm_t(a, b):
    return jax.lax.dot_general(a.astype(_BF16), b.astype(_BF16),
                               (((1,), (1,)), ((), ())),
                               preferred_element_type=_F32)


def _rep64(a):
    return _mm(a, _expand_k())


def _leaky(v):
    return jnp.where(v >= 0, v, 0.2 * v)


def _gat_layer(h_prox, h_nodes, asd, need_prox):
    sc_prox = _mm(h_prox, asd)
    sc_nodes = _mm(h_nodes, asd)
    as_prox, ad_prox = sc_prox[:, :HD], sc_prox[:, HD:]
    as_nodes, ad_nodes = sc_nodes[:, :HD], sc_nodes[:, HD:]

    r1 = _rep64(as_prox)
    v = jnp.sum(r1 * _sel_p().astype(_F32), axis=0, keepdims=True)
    e_node = _leaky(v + _rep64(ad_nodes))
    ex_node = jnp.exp(e_node)
    ex_self_n = jnp.exp(_leaky(as_nodes + ad_nodes))
    denom_n = _mm(ex_node, _blocksum_m()) + ex_self_n + 1e-16
    h_bd = (_mm(_sel_p_t(), h_prox) * _blockmask(_F32))
    out_nodes = ((_mm(ex_node, h_bd) + _rep64(ex_self_n) * h_nodes)
                 / _rep64(denom_n))

    if not need_prox:
        return None, out_nodes

    as_stack = _mm_t(_blocksum_m(), as_nodes)
    rowsel = jnp.where(_iota((D, P), 0) % OC == _iota((D, P), 1),
                       1.0, 0.0).astype(_F32)
    y_d = _mm_t(_blocksum_m(), ad_prox)
    ad_prox_col = jnp.sum(y_d * rowsel, axis=1, keepdims=True)
    y_s = _mm_t(_blocksum_m(), as_prox)
    as_prox_col = jnp.sum(y_s * rowsel, axis=1, keepdims=True)

    ex_prox = jnp.exp(_leaky(as_stack + ad_prox_col))
    ex_self_p = jnp.exp(_leaky(as_prox_col + ad_prox_col))
    denom_p = (jnp.sum(ex_prox, axis=1, keepdims=True)
               + ex_self_p + 1e-16)
    r_full = _mm(ex_prox, h_nodes) / denom_p
    out_prox = _mm(_sel_p(), r_full * _blockmask(_F32))
    alpha_self_col = ex_self_p / denom_p
    rep_self = _mm(_sel_p(), alpha_self_col * _blockmask(_F32))
    out_prox = out_prox + rep_self * h_prox
    return out_prox, out_nodes


def _att_blockdiag_in(att_s, att_d):
    sel_k = _blocksum_m()
    rowsel = jnp.where(_iota((D, OC), 0) % OC == _iota((D, OC), 1),
                       1.0, 0.0).astype(_F32)
    s_col = jnp.sum(_mm(sel_k, att_s) * rowsel, axis=1, keepdims=True)
    d_col = jnp.sum(_mm(sel_k, att_d) * rowsel, axis=1, keepdims=True)
    sel_f = sel_k.astype(_F32)
    return jnp.concatenate([s_col * sel_f, d_col * sel_f], axis=1)


def _model_body(x_ref, prox_ref, W1_ref,
                as1_ref, ad1_ref, as2_ref, ad2_ref,
                W2_ref, Wm1_ref, Wm2_ref, Wf_ref,
                b1_ref, b2_ref, bm1_ref, bm2_ref, bf_ref,
                preds_ref, feats_ref,
                x_s, w1_s, w2_s, wm1_s, wm2_s, wf_s, f3_s, preds_s,
                sem_x, sem_w1, sem_w2, sem_wm1, sem_wm2, sem_wf,
                sem_f3, sem_p1, sem_p2):
    cp_x = pltpu.make_async_copy(x_ref, x_s, sem_x)
    cp_w1 = pltpu.make_async_copy(W1_ref, w1_s, sem_w1)
    cp_w2 = pltpu.make_async_copy(W2_ref, w2_s, sem_w2)
    cp_wm1 = pltpu.make_async_copy(Wm1_ref, wm1_s, sem_wm1)
    cp_wm2 = pltpu.make_async_copy(Wm2_ref, wm2_s, sem_wm2)
    cp_wf = pltpu.make_async_copy(Wf_ref, wf_s, sem_wf)
    cp_x.start()
    cp_w1.start()
    cp_w2.start()
    cp_wm1.start()
    cp_wm2.start()
    cp_wf.start()

    asd1 = _att_blockdiag_in(as1_ref[...], ad1_ref[...])
    asd2 = _att_blockdiag_in(as2_ref[...], ad2_ref[...])
    cp_w1.wait()
    h1_prox = _mm(prox_ref[...], w1_s[...])
    cp_x.wait()
    h1_nodes = _mm(x_s[...], w1_s[...])
    p1, n1 = _gat_layer(h1_prox, h1_nodes, asd1, need_prox=True)
    f1_prox = jax.nn.relu(p1 + b1_ref[...])
    f1_nodes = jax.nn.relu(n1 + b1_ref[...])

    cp_w2.wait()
    h2_prox = _mm(f1_prox, w2_s[...])
    h2_nodes = _mm(f1_nodes, w2_s[...])
    _, n2 = _gat_layer(h2_prox, h2_nodes, asd2, need_prox=False)
    f2 = jax.nn.relu(n2 + b2_ref[...])

    cp_wm1.wait()
    hmid = jax.nn.relu(_mm(f2, wm1_s[...]) + bm1_ref[...])
    cp_wm2.wait()
    f3 = jax.nn.relu(_mm(hmid, wm2_s[...]) + bm2_ref[...])
    f3_s[...] = f3
    cp_f3 = pltpu.make_async_copy(f3_s, feats_ref, sem_f3)
    cp_f3.start()

    cp_wf.wait()
    fh = f3.astype(_BF16)
    wfb = wf_s[...].astype(_BF16)
    preds_s[0:NH, :] = jax.lax.dot_general(
        fh[0:NH, :], wfb, (((1,), (0,)), ((), ())),
        preferred_element_type=_F32) + bf_ref[...]
    cp_p1 = pltpu.make_async_copy(preds_s.at[0:NH, :],
                                  preds_ref.at[0:NH, :], sem_p1)
    cp_p1.start()
    preds_s[NH:N, :] = jax.lax.dot_general(
        fh[NH:N, :], wfb, (((1,), (0,)), ((), ())),
        preferred_element_type=_F32) + bf_ref[...]
    cp_p2 = pltpu.make_async_copy(preds_s.at[NH:N, :],
                                  preds_ref.at[NH:N, :], sem_p2)
    cp_p2.start()
    cp_f3.wait()
    cp_p1.wait()
    cp_p2.wait()


def kernel(x, proxies, W1, as1, ad1, b1, W2, as2, ad2, b2,
           Wm1, bm1, Wm2, bm2, Wf, bf):
    out_shape = (jax.ShapeDtypeStruct((N, D), _F32),
                 jax.ShapeDtypeStruct((N, D), _F32))
    vm = pl.BlockSpec(memory_space=pltpu.VMEM)
    hbm = pl.BlockSpec(memory_space=pltpu.HBM)
    preds, feats = pl.pallas_call(
        _model_body,
        out_shape=out_shape,
        in_specs=[hbm, vm, hbm,
                  vm, vm, vm, vm,
                  hbm, hbm, hbm, hbm,
                  vm, vm, vm, vm, vm],
        out_specs=(hbm, hbm),
        scratch_shapes=[
            pltpu.VMEM((N, D), _F32),
            pltpu.VMEM((D, D), _F32),
            pltpu.VMEM((D, D), _F32),
            pltpu.VMEM((D, 4 * D), _F32),
            pltpu.VMEM((4 * D, D), _F32),
            pltpu.VMEM((D, D), _F32),
            pltpu.VMEM((N, D), _F32),
            pltpu.VMEM((N, D), _F32),
            pltpu.SemaphoreType.DMA,
            pltpu.SemaphoreType.DMA,
            pltpu.SemaphoreType.DMA,
            pltpu.SemaphoreType.DMA,
            pltpu.SemaphoreType.DMA,
            pltpu.SemaphoreType.DMA,
            pltpu.SemaphoreType.DMA,
            pltpu.SemaphoreType.DMA,
            pltpu.SemaphoreType.DMA,
        ],
    )(x, proxies, W1,
      as1, ad1, as2, ad2,
      W2, Wm1, Wm2, Wf,
      b1.reshape(1, D), b2.reshape(1, D),
      bm1.reshape(1, 4 * D), bm2.reshape(1, D), bf.reshape(1, D))
    return preds, feats
